# baseline (speedup 1.0000x reference)
import jax
import jax.numpy as jnp
from jax import lax
from jax.experimental import pallas as pl
from jax.experimental.pallas import tpu as pltpu
from jax.experimental.pallas import tpu_sc as plsc

_NC, _NS, _L = 2, 16, 16
_NW = _NC * _NS

_B, _SEQ, _D = 4, 4096, 1024
_ROWS_PER_W = _SEQ // _NW
_CH = 16
_NCH = _ROWS_PER_W // _CH
_UNROLL = 8
_JITERS = _D // (_L * _UNROLL)


def _sc_body(x_hbm, pos_hbm, out_hbm,
             xv0, xv1, pv0, pv1,
             xs0, xs1, os0, os1, ps0, ps1):
    wid = lax.axis_index("s") * _NC + lax.axis_index("c")
    base_s = wid * _ROWS_PER_W
    xbufs, pbufs = [xv0, xv1], [pv0, pv1]
    xsems, osems, psems = [xs0, xs1], [os0, os1], [ps0, ps1]

    units = [(c, b) for c in range(_NCH) for b in range(_B)]

    pos_cp = [None, None]
    x_cp = [None, None]
    st_cp = [None, None]
    pos_cp[0] = pltpu.async_copy(
        pos_hbm.at[pl.ds(base_s, _CH)], pv0, ps0)
    x_cp[0] = pltpu.async_copy(
        x_hbm.at[0, pl.ds(base_s, _CH)], xv0, xs0)

    for i, (c, b) in enumerate(units):
        nb = i % 2
        if i + 1 < len(units):
            onb = (i + 1) % 2
            if st_cp[onb] is not None:
                st_cp[onb].wait()
                st_cp[onb] = None
            nc, nbatch = units[i + 1]
            x_cp[onb] = pltpu.async_copy(
                x_hbm.at[nbatch, pl.ds(base_s + nc * _CH, _CH)],
                xbufs[onb], xsems[onb])
        if b == 0 and c + 1 < _NCH:
            pos_cp[(c + 1) % 2] = pltpu.async_copy(
                pos_hbm.at[pl.ds(base_s + (c + 1) * _CH, _CH)],
                pbufs[(c + 1) % 2], psems[(c + 1) % 2])
        if b == 0:
            pos_cp[c % 2].wait()
        x_cp[nb].wait()

        xv, pv = xbufs[nb], pbufs[c % 2]

        def row_body(r, carry):
            def col_body(j, carry2):
                v0 = j * (_L * _UNROLL)
                for u in range(_UNROLL):
                    sl = pl.ds(v0 + u * _L, _L)
                    xv[r, sl] = xv[r, sl] + pv[r, sl]
                return carry2

            lax.fori_loop(0, _JITERS, col_body, 0)
            return carry

        lax.fori_loop(0, _CH, row_body, 0)
        st_cp[nb] = pltpu.async_copy(
            xbufs[nb], out_hbm.at[b, pl.ds(base_s + c * _CH, _CH)], osems[nb])

    for cp in st_cp:
        if cp is not None:
            cp.wait()


def kernel(x, pos_table):
    batch, seq_len, d_model = x.shape
    k = pl.kernel(
        _sc_body,
        out_type=jax.ShapeDtypeStruct((batch, seq_len, d_model), x.dtype),
        mesh=plsc.VectorSubcoreMesh(core_axis_name="c", subcore_axis_name="s"),
        scratch_types=[
            pltpu.VMEM((_CH, _D), jnp.float32),
            pltpu.VMEM((_CH, _D), jnp.float32),
            pltpu.VMEM((_CH, _D), jnp.float32),
            pltpu.VMEM((_CH, _D), jnp.float32),
            pltpu.SemaphoreType.DMA,
            pltpu.SemaphoreType.DMA,
            pltpu.SemaphoreType.DMA,
            pltpu.SemaphoreType.DMA,
            pltpu.SemaphoreType.DMA,
            pltpu.SemaphoreType.DMA,
        ],
    )
    return k(x, pos_table)


# speedup vs baseline: 2.0646x; 2.0646x over previous
import jax
import jax.numpy as jnp
from jax import lax
from jax.experimental import pallas as pl
from jax.experimental.pallas import tpu as pltpu
from jax.experimental.pallas import tpu_sc as plsc

_NC, _NS, _L = 2, 16, 16
_NW = _NC * _NS

_B, _SEQ, _D = 4, 4096, 1024
_ROWS_PER_W = _SEQ // _NW
_CH = 16
_NCH = _ROWS_PER_W // _CH
_UNROLL = 8
_JITERS = _D // (_L * _UNROLL)


def _sc_body(x_hbm, pos_hbm, out_hbm,
             xv0, xv1, pv0, pv1,
             xs0, xs1, os0, os1, ps0, ps1):
    wid = lax.axis_index("s") * _NC + lax.axis_index("c")
    base_s = wid * _ROWS_PER_W
    xbufs, pbufs = [xv0, xv1], [pv0, pv1]
    xsems, osems, psems = [xs0, xs1], [os0, os1], [ps0, ps1]

    units = [(c, b) for c in range(_NCH) for b in range(_B)]

    pos_cp = [None, None]
    x_cp = [None, None]
    st_cp = [None, None]
    pos_cp[0] = pltpu.async_copy(
        pos_hbm.at[pl.ds(base_s, _CH)], pv0, ps0)
    x_cp[0] = pltpu.async_copy(
        x_hbm.at[0, pl.ds(base_s, _CH)], xv0, xs0)

    for i, (c, b) in enumerate(units):
        nb = i % 2
        if i + 1 < len(units):
            onb = (i + 1) % 2
            if st_cp[onb] is not None:
                st_cp[onb].wait()
                st_cp[onb] = None
            nc, nbatch = units[i + 1]
            x_cp[onb] = pltpu.async_copy(
                x_hbm.at[nbatch, pl.ds(base_s + nc * _CH, _CH)],
                xbufs[onb], xsems[onb])
        if b == 0 and c + 1 < _NCH:
            pos_cp[(c + 1) % 2] = pltpu.async_copy(
                pos_hbm.at[pl.ds(base_s + (c + 1) * _CH, _CH)],
                pbufs[(c + 1) % 2], psems[(c + 1) % 2])
        if b == 0:
            pos_cp[c % 2].wait()
        x_cp[nb].wait()

        xv, pv = xbufs[nb], pbufs[c % 2]

        def col_body(j, carry):
            sl = pl.ds(j * _L, _L)
            for r in range(_CH):
                xv[r, sl] = xv[r, sl] + pv[r, sl]
            return carry

        lax.fori_loop(0, _D // _L, col_body, 0)
        st_cp[nb] = pltpu.async_copy(
            xbufs[nb], out_hbm.at[b, pl.ds(base_s + c * _CH, _CH)], osems[nb])

    for cp in st_cp:
        if cp is not None:
            cp.wait()


def kernel(x, pos_table):
    batch, seq_len, d_model = x.shape
    k = pl.kernel(
        _sc_body,
        out_type=jax.ShapeDtypeStruct((batch, seq_len, d_model), x.dtype),
        mesh=plsc.VectorSubcoreMesh(core_axis_name="c", subcore_axis_name="s"),
        scratch_types=[
            pltpu.VMEM((_CH, _D), jnp.float32),
            pltpu.VMEM((_CH, _D), jnp.float32),
            pltpu.VMEM((_CH, _D), jnp.float32),
            pltpu.VMEM((_CH, _D), jnp.float32),
            pltpu.SemaphoreType.DMA,
            pltpu.SemaphoreType.DMA,
            pltpu.SemaphoreType.DMA,
            pltpu.SemaphoreType.DMA,
            pltpu.SemaphoreType.DMA,
            pltpu.SemaphoreType.DMA,
        ],
    )
    return k(x, pos_table)
